# Initial kernel scaffold; baseline (speedup 1.0000x reference)
import jax
import jax.numpy as jnp
from jax.experimental import pallas as pl


def kernel(x, edge_weight, W1, b1, W2, b2, W3, b3, Wmu, bmu, Wlv, blv, Wd1, bd1, Wd2, bd2, edge_index, beta):
    raise NotImplementedError("write your pallas kernel here")


# speedup vs baseline: 9.4931x; 9.4931x over previous
import functools

import numpy as np
import jax
import jax.numpy as jnp
from jax import lax
from jax.experimental import pallas as pl
from jax.experimental.pallas import tpu as pltpu
from jax.experimental.pallas import tpu_sc as plsc

N = 10000
E = 320000
NP = 10240
NC, NS = 2, 16
NW = NC * NS
K = 80
NB = E // (NW * K)
RT = NP // NS
FC = 128
BR = 1000
SLABS = 5
SB = NB // SLABS


_MESH = plsc.VectorSubcoreMesh(
    core_axis_name="c", subcore_axis_name="s", num_cores=NC, num_subcores=NS)


@functools.partial(
    pl.kernel,
    out_type=jax.ShapeDtypeStruct((NC, NP), jnp.float32),
    mesh=_MESH,
    compiler_params=pltpu.CompilerParams(needs_layout_passes=False),
    scratch_types=[
        pltpu.VMEM((NB, K), jnp.int32),
        pltpu.VMEM((NB, K), jnp.float32),
        pltpu.VMEM((NP // NS,), jnp.float32),
        pltpu.VMEM_SHARED((NP,), jnp.float32),
    ],
)
def _deg_kernel(dst_hbm, ew_hbm, out_hbm, dst_v, ew_v, tbuf, acc):
    c = lax.axis_index("c")
    s = lax.axis_index("s")
    w = s * NC + c
    pltpu.sync_copy(dst_hbm.at[w], dst_v)
    pltpu.sync_copy(ew_hbm.at[w], ew_v)

    def zb(i, _):
        tbuf[pl.ds(i * 16, 16)] = jnp.zeros((16,), jnp.float32)
        return 0

    lax.fori_loop(0, (NP // NS) // 16, zb, 0)
    pltpu.sync_copy(tbuf, acc.at[pl.ds(s * (NP // NS), NP // NS)])
    plsc.subcore_barrier()

    def batch(j, _):
        pltpu.sync_copy(ew_v.at[j], acc.at[dst_v.at[j]], add=True)
        return 0

    lax.fori_loop(0, NB, batch, 0)
    plsc.subcore_barrier()
    pltpu.sync_copy(acc.at[pl.ds(s * (NP // NS), NP // NS)], tbuf)
    pltpu.sync_copy(tbuf, out_hbm.at[c, pl.ds(s * (NP // NS), NP // NS)])


@functools.partial(
    pl.kernel,
    out_type=jax.ShapeDtypeStruct((NW, NB, K), jnp.float32),
    mesh=_MESH,
    compiler_params=pltpu.CompilerParams(needs_layout_passes=False),
    scratch_types=[
        pltpu.VMEM((NB, K), jnp.int32),
        pltpu.VMEM((NB, K), jnp.int32),
        pltpu.VMEM((NB, K), jnp.float32),
        pltpu.VMEM((NB, K), jnp.float32),
        pltpu.VMEM((NP,), jnp.float32),
    ],
)
def _norm_kernel(src_hbm, dst_hbm, ew_hbm, dis_hbm, out_hbm,
                 src_v, dst_v, ew_v, nrm_v, dis_v):
    c = lax.axis_index("c")
    s = lax.axis_index("s")
    w = s * NC + c
    pltpu.sync_copy(src_hbm.at[w], src_v)
    pltpu.sync_copy(dst_hbm.at[w], dst_v)
    pltpu.sync_copy(ew_hbm.at[w], ew_v)
    pltpu.sync_copy(dis_hbm, dis_v)

    def batch(j, _):
        for t in range(K // 16):
            sl = pl.ds(t * 16, 16)
            g1 = plsc.load_gather(dis_v, [src_v[j, sl]])
            g2 = plsc.load_gather(dis_v, [dst_v[j, sl]])
            nrm_v[j, sl] = ew_v[j, sl] * g1 * g2
        return 0

    lax.fori_loop(0, NB, batch, 0)
    pltpu.sync_copy(nrm_v, out_hbm.at[w])


@functools.partial(
    pl.kernel,
    out_type=jax.ShapeDtypeStruct((NC, NP, FC), jnp.float32),
    mesh=_MESH,
    compiler_params=pltpu.CompilerParams(needs_layout_passes=False),
    scratch_types=[
        pltpu.VMEM((SB, K), jnp.int32),
        pltpu.VMEM((SB, K), jnp.int32),
        pltpu.VMEM((SB, K), jnp.float32),
        pltpu.VMEM((K, FC), jnp.float32),
        pltpu.VMEM((RT // 5, FC), jnp.float32),
        pltpu.VMEM_SHARED((NP, FC), jnp.float32),
        pltpu.SemaphoreType.DMA,
    ],
)
def _spmv_kernel(x_hbm, src_hbm, dst_hbm, nrm_hbm, out_hbm,
                 src_v, dst_v, nrm_v, rows_v, wbuf, acc, sem):
    c = lax.axis_index("c")
    s = lax.axis_index("s")
    w = s * NC + c

    def zb(i, _):
        for t in range(FC // 16):
            wbuf[i, pl.ds(t * 16, 16)] = jnp.zeros((16,), jnp.float32)
        return 0

    lax.fori_loop(0, RT // 5, zb, 0)
    for q in range(5):
        pltpu.sync_copy(wbuf, acc.at[pl.ds(s * RT + q * (RT // 5), RT // 5)])
    plsc.subcore_barrier()

    def slab(m, _):
        pltpu.sync_copy(src_hbm.at[w, m], src_v)
        pltpu.sync_copy(dst_hbm.at[w, m], dst_v)
        pltpu.sync_copy(nrm_hbm.at[w, m], nrm_v)

        def batch(j, _):
            pltpu.async_copy(x_hbm.at[src_v.at[j]], rows_v, sem).wait()

            def scale16(g, _):
                nv16 = nrm_v[j, pl.ds(g * 16, 16)]
                for l in range(16):
                    nv = nv16[l]
                    k = g * 16 + l
                    for t in range(FC // 16):
                        sl = pl.ds(t * 16, 16)
                        rows_v[k, sl] = rows_v[k, sl] * nv
                return 0

            lax.fori_loop(0, K // 16, scale16, 0)
            pltpu.sync_copy(rows_v, acc.at[dst_v.at[j]], add=True)
            return 0

        lax.fori_loop(0, SB, batch, 0)
        return 0

    lax.fori_loop(0, SLABS, slab, 0)
    plsc.subcore_barrier()
    for q in range(5):
        pltpu.sync_copy(acc.at[pl.ds(s * RT + q * (RT // 5), RT // 5)], wbuf)
        pltpu.sync_copy(wbuf, out_hbm.at[c, pl.ds(s * RT + q * (RT // 5), RT // 5)])


def _dis(deg2):
    def body(deg_ref, out_ref):
        d = deg_ref[0, :] + deg_ref[1, :] + 1.0
        out_ref[0, :] = lax.rsqrt(d)

    return pl.pallas_call(
        body, out_shape=jax.ShapeDtypeStruct((1, NP), jnp.float32)
    )(deg2)


def _row_block_specs(width):
    return pl.BlockSpec((BR, width), lambda i: (i, 0))


def _mm1(acc0, acc1, x, dis_col, W1, b1):
    def body(a0, a1, x_r, dc, w_r, b_r, o_r):
        d2 = dc[...] * dc[...]
        xc = a0[...] + a1[...] + d2 * x_r[...]
        h = jnp.dot(xc, w_r[...], preferred_element_type=jnp.float32) + b_r[...]
        nrm = jnp.sqrt(jnp.sum(h * h, axis=1, keepdims=True))
        h = h / jnp.maximum(nrm, 1e-12)
        o_r[...] = jnp.maximum(h, 0.0)

    fo = W1.shape[1]
    return pl.pallas_call(
        body,
        grid=(N // BR,),
        in_specs=[
            _row_block_specs(FC), _row_block_specs(FC), _row_block_specs(FC),
            pl.BlockSpec((BR, 1), lambda i: (i, 0)),
            pl.BlockSpec((FC, fo), lambda i: (0, 0)),
            pl.BlockSpec((1, fo), lambda i: (0, 0)),
        ],
        out_specs=pl.BlockSpec((BR, fo), lambda i: (i, 0)),
        out_shape=jax.ShapeDtypeStruct((N, fo), jnp.float32),
    )(acc0, acc1, x, dis_col, W1, b1)


def _mm(h, W):
    fi, fo = W.shape

    def body(h_r, w_r, o_r):
        o_r[...] = jnp.dot(h_r[...], w_r[...], preferred_element_type=jnp.float32)

    return pl.pallas_call(
        body,
        grid=(N // BR,),
        in_specs=[
            _row_block_specs(fi),
            pl.BlockSpec((fi, fo), lambda i: (0, 0)),
        ],
        out_specs=pl.BlockSpec((BR, fo), lambda i: (i, 0)),
        out_shape=jax.ShapeDtypeStruct((N, fo), jnp.float32),
    )(h, W)


def _comb(acc0, acc1, t, dis_col, b):
    fo = t.shape[1]

    def body(a0, a1, t_r, dc, b_r, o_r):
        d2 = dc[...] * dc[...]
        h = a0[...] + a1[...] + d2 * t_r[...] + b_r[...]
        nrm = jnp.sqrt(jnp.sum(h * h, axis=1, keepdims=True))
        h = h / jnp.maximum(nrm, 1e-12)
        o_r[...] = jnp.maximum(h, 0.0)

    return pl.pallas_call(
        body,
        grid=(N // BR,),
        in_specs=[
            _row_block_specs(fo), _row_block_specs(fo), _row_block_specs(fo),
            pl.BlockSpec((BR, 1), lambda i: (i, 0)),
            pl.BlockSpec((1, fo), lambda i: (0, 0)),
        ],
        out_specs=pl.BlockSpec((BR, fo), lambda i: (i, 0)),
        out_shape=jax.ShapeDtypeStruct((N, fo), jnp.float32),
    )(acc0, acc1, t, dis_col, b)


def _head(h3, Wmu, bmu, Wlv, blv, eps, beta_arr):
    fi = h3.shape[1]
    fo = Wmu.shape[1]

    def body(h_r, wm, bm, wl, bl, e_r, bet, mu_r, lv_r, pool_r):
        i = pl.program_id(0)
        mu = jnp.dot(h_r[...], wm[...], preferred_element_type=jnp.float32) + bm[...]
        lv = jnp.dot(h_r[...], wl[...], preferred_element_type=jnp.float32) + bl[...]
        mu_r[...] = mu
        lv_r[...] = lv
        std = jnp.exp(0.5 * (bet[0, 0] * lv))
        z = mu + e_r[...] * std
        bmax = jnp.max(z, axis=0, keepdims=True)
        bsum = jnp.sum(z, axis=0, keepdims=True)

        @pl.when(i == 0)
        def _():
            pool_r[0:1, :] = bmax
            pool_r[1:2, :] = bsum

        @pl.when(i > 0)
        def _():
            pool_r[0:1, :] = jnp.maximum(pool_r[0:1, :], bmax)
            pool_r[1:2, :] = pool_r[1:2, :] + bsum

    return pl.pallas_call(
        body,
        grid=(N // BR,),
        in_specs=[
            _row_block_specs(fi),
            pl.BlockSpec((fi, fo), lambda i: (0, 0)),
            pl.BlockSpec((1, fo), lambda i: (0, 0)),
            pl.BlockSpec((fi, fo), lambda i: (0, 0)),
            pl.BlockSpec((1, fo), lambda i: (0, 0)),
            _row_block_specs(fo),
            pl.BlockSpec(memory_space=pltpu.SMEM),
        ],
        out_specs=[
            pl.BlockSpec((BR, fo), lambda i: (i, 0)),
            pl.BlockSpec((BR, fo), lambda i: (i, 0)),
            pl.BlockSpec((2, fo), lambda i: (0, 0)),
        ],
        out_shape=[
            jax.ShapeDtypeStruct((N, fo), jnp.float32),
            jax.ShapeDtypeStruct((N, fo), jnp.float32),
            jax.ShapeDtypeStruct((2, fo), jnp.float32),
        ],
    )(h3, Wmu, bmu, Wlv, blv, eps, beta_arr)


def _dec(pool, Wd1, bd1, Wd2, bd2):
    def body(p_r, w1, b1_r, w2, b2_r, o_r):
        zmax = p_r[0:1, :]
        zmean = p_r[1:2, :] * (1.0 / N)
        rz = jnp.concatenate([zmax, zmean], axis=1)
        h = jnp.dot(rz, w1[...], preferred_element_type=jnp.float32) + b1_r[...]
        h = jnp.maximum(h, 0.0)
        o = jnp.dot(h, w2[...], preferred_element_type=jnp.float32) + b2_r[...]
        o_r[...] = jax.nn.sigmoid(o)

    return pl.pallas_call(
        body, out_shape=jax.ShapeDtypeStruct((1, Wd2.shape[1]), jnp.float32)
    )(pool, Wd1, bd1, Wd2, bd2)


def kernel(x, edge_weight, W1, b1, W2, b2, W3, b3, Wmu, bmu, Wlv, blv,
           Wd1, bd1, Wd2, bd2, edge_index, beta):
    src = edge_index[0].reshape(NW, NB, K)
    dst = edge_index[1].reshape(NW, NB, K)
    eww = edge_weight.reshape(NW, NB, K)

    deg2 = _deg_kernel(dst, eww)
    dis_row = _dis(deg2)
    dis_flat = dis_row.reshape(NP)
    dis_col = dis_flat[:N].reshape(N, 1)
    nrm = _norm_kernel(src, dst, eww, dis_flat)

    src4 = src.reshape(NW, SLABS, SB, K)
    dst4 = dst.reshape(NW, SLABS, SB, K)
    nrm4 = nrm.reshape(NW, SLABS, SB, K)

    acc1 = _spmv_kernel(x, src4, dst4, nrm4)[:, :N, :]
    h1 = _mm1(acc1[0], acc1[1], x, dis_col, W1, b1.reshape(1, -1))

    t2 = _mm(h1, W2)
    acc2 = jnp.concatenate(
        [_spmv_kernel(t2[:, i * FC:(i + 1) * FC], src4, dst4, nrm4)[:, :N, :]
         for i in range(4)], axis=2)
    h2 = _comb(acc2[0], acc2[1], t2, dis_col, b2.reshape(1, -1))

    t3 = _mm(h2, W3)
    acc3 = jnp.concatenate(
        [_spmv_kernel(t3[:, i * FC:(i + 1) * FC], src4, dst4, nrm4)[:, :N, :]
         for i in range(2)], axis=2)
    h3 = _comb(acc3[0], acc3[1], t3, dis_col, b3.reshape(1, -1))

    beta_arr = jnp.asarray(beta, jnp.float32).reshape(1, 1)
    eps = jax.random.normal(jax.random.key(42), (N, 512), jnp.float32) * 0.01
    mu, logvar, pool = _head(h3, Wmu, bmu.reshape(1, -1), Wlv, blv.reshape(1, -1),
                             eps, beta_arr)
    recon = _dec(pool, Wd1, bd1.reshape(1, -1), Wd2, bd2.reshape(1, -1))
    return (recon, mu, logvar)


# speedup vs baseline: 15.6401x; 1.6475x over previous
import functools

import numpy as np
import jax
import jax.numpy as jnp
from jax import lax
from jax.experimental import pallas as pl
from jax.experimental.pallas import tpu as pltpu
from jax.experimental.pallas import tpu_sc as plsc

N = 10000
E = 320000
NP = 10240
NC, NS = 2, 16
NW = NC * NS
K = 80
NB = E // (NW * K)
RT = NP // NS
FC = 128
BR = 1000
SLABS = 5
SB = NB // SLABS


_MESH = plsc.VectorSubcoreMesh(
    core_axis_name="c", subcore_axis_name="s", num_cores=NC, num_subcores=NS)


@functools.partial(
    pl.kernel,
    out_type=jax.ShapeDtypeStruct((NC, NP), jnp.float32),
    mesh=_MESH,
    compiler_params=pltpu.CompilerParams(needs_layout_passes=False),
    scratch_types=[
        pltpu.VMEM((NB, K), jnp.int32),
        pltpu.VMEM((NB, K), jnp.float32),
        pltpu.VMEM((NP // NS,), jnp.float32),
        pltpu.VMEM_SHARED((NP,), jnp.float32),
    ],
)
def _deg_kernel(dst_hbm, ew_hbm, out_hbm, dst_v, ew_v, tbuf, acc):
    c = lax.axis_index("c")
    s = lax.axis_index("s")
    w = s * NC + c
    pltpu.sync_copy(dst_hbm.at[w], dst_v)
    pltpu.sync_copy(ew_hbm.at[w], ew_v)

    def zb(i, _):
        tbuf[pl.ds(i * 16, 16)] = jnp.zeros((16,), jnp.float32)
        return 0

    lax.fori_loop(0, (NP // NS) // 16, zb, 0)
    pltpu.sync_copy(tbuf, acc.at[pl.ds(s * (NP // NS), NP // NS)])
    plsc.subcore_barrier()

    def batch(j, _):
        pltpu.sync_copy(ew_v.at[j], acc.at[dst_v.at[j]], add=True)
        return 0

    lax.fori_loop(0, NB, batch, 0)
    plsc.subcore_barrier()
    pltpu.sync_copy(acc.at[pl.ds(s * (NP // NS), NP // NS)], tbuf)
    pltpu.sync_copy(tbuf, out_hbm.at[c, pl.ds(s * (NP // NS), NP // NS)])


@functools.partial(
    pl.kernel,
    out_type=jax.ShapeDtypeStruct((NW, NB, K), jnp.float32),
    mesh=_MESH,
    compiler_params=pltpu.CompilerParams(needs_layout_passes=False),
    scratch_types=[
        pltpu.VMEM((NB, K), jnp.int32),
        pltpu.VMEM((NB, K), jnp.int32),
        pltpu.VMEM((NB, K), jnp.float32),
        pltpu.VMEM((NB, K), jnp.float32),
        pltpu.VMEM((NP,), jnp.float32),
    ],
)
def _norm_kernel(src_hbm, dst_hbm, ew_hbm, dis_hbm, out_hbm,
                 src_v, dst_v, ew_v, nrm_v, dis_v):
    c = lax.axis_index("c")
    s = lax.axis_index("s")
    w = s * NC + c
    pltpu.sync_copy(src_hbm.at[w], src_v)
    pltpu.sync_copy(dst_hbm.at[w], dst_v)
    pltpu.sync_copy(ew_hbm.at[w], ew_v)
    pltpu.sync_copy(dis_hbm, dis_v)

    def batch(j, _):
        for t in range(K // 16):
            sl = pl.ds(t * 16, 16)
            g1 = plsc.load_gather(dis_v, [src_v[j, sl]])
            g2 = plsc.load_gather(dis_v, [dst_v[j, sl]])
            nrm_v[j, sl] = ew_v[j, sl] * g1 * g2
        return 0

    lax.fori_loop(0, NB, batch, 0)
    pltpu.sync_copy(nrm_v, out_hbm.at[w])


@functools.partial(
    pl.kernel,
    out_type=jax.ShapeDtypeStruct((NC, NP, FC), jnp.float32),
    mesh=_MESH,
    compiler_params=pltpu.CompilerParams(needs_layout_passes=False),
    scratch_types=[
        pltpu.VMEM((SB, K), jnp.int32),
        pltpu.VMEM((SB, K), jnp.int32),
        pltpu.VMEM((SB, K), jnp.float32),
        pltpu.VMEM((K, FC), jnp.float32),
        pltpu.VMEM((K, FC), jnp.float32),
        pltpu.VMEM((K, FC), jnp.float32),
        pltpu.VMEM((RT // 20, FC), jnp.float32),
        pltpu.VMEM_SHARED((NP, FC), jnp.float32),
        pltpu.SemaphoreType.DMA,
        pltpu.SemaphoreType.DMA,
        pltpu.SemaphoreType.DMA,
    ],
)
def _spmv_kernel(x_hbm, src_hbm, dst_hbm, nrm_hbm, out_hbm,
                 src_v, dst_v, nrm_v, r0, r1, r2, wbuf, acc,
                 g0, g1, g2):
    c = lax.axis_index("c")
    s = lax.axis_index("s")
    w = s * NC + c
    rows = (r0, r1, r2)
    gsem = (g0, g1, g2)

    def zb(i, _):
        for t in range(FC // 16):
            wbuf[i, pl.ds(t * 16, 16)] = jnp.zeros((16,), jnp.float32)
        return 0

    lax.fori_loop(0, RT // 20, zb, 0)
    for q in range(20):
        pltpu.sync_copy(wbuf, acc.at[pl.ds(s * RT + q * (RT // 20), RT // 20)])
    plsc.subcore_barrier()

    def scale(j, buf):
        def scale16(g, _):
            nv16 = nrm_v[j, pl.ds(g * 16, 16)]
            for l in range(16):
                nv = nv16[l]
                k = g * 16 + l
                for t in range(FC // 16):
                    sl = pl.ds(t * 16, 16)
                    buf[k, sl] = buf[k, sl] * nv
            return 0

        lax.fori_loop(0, K // 16, scale16, 0)

    def slab(m, _):
        pltpu.sync_copy(src_hbm.at[w, m], src_v)
        pltpu.sync_copy(dst_hbm.at[w, m], dst_v)
        pltpu.sync_copy(nrm_hbm.at[w, m], nrm_v)
        pltpu.async_copy(x_hbm.at[src_v.at[0]], r0, g0)
        pltpu.async_copy(x_hbm.at[src_v.at[1]], r1, g1)

        def super_batch(q, _):
            for b in range(3):
                j = q * 3 + b
                buf = rows[b]
                nb = (b + 2) % 3
                pltpu.make_async_copy(x_hbm.at[src_v.at[j]], buf, gsem[b]).wait()
                nxt = j + 2

                @pl.when(nxt < SB)
                def _():
                    pltpu.async_copy(x_hbm.at[src_v.at[nxt]], rows[nb], gsem[nb])

                scale(j, buf)
                pltpu.sync_copy(buf, acc.at[dst_v.at[j]], add=True)
            return 0

        lax.fori_loop(0, SB // 3, super_batch, 0)
        pltpu.make_async_copy(x_hbm.at[src_v.at[SB - 1]], r0, g0).wait()
        scale(SB - 1, r0)
        pltpu.sync_copy(r0, acc.at[dst_v.at[SB - 1]], add=True)
        return 0

    lax.fori_loop(0, SLABS, slab, 0)
    plsc.subcore_barrier()
    for q in range(20):
        pltpu.sync_copy(acc.at[pl.ds(s * RT + q * (RT // 20), RT // 20)], wbuf)
        pltpu.sync_copy(wbuf, out_hbm.at[c, pl.ds(s * RT + q * (RT // 20), RT // 20)])


def _dis(deg2):
    def body(deg_ref, out_ref):
        d = deg_ref[0, :] + deg_ref[1, :] + 1.0
        out_ref[0, :] = lax.rsqrt(d)

    return pl.pallas_call(
        body, out_shape=jax.ShapeDtypeStruct((1, NP), jnp.float32)
    )(deg2)


def _row_block_specs(width):
    return pl.BlockSpec((BR, width), lambda i: (i, 0))


def _mm1(acc0, acc1, x, dis_col, W1, b1):
    def body(a0, a1, x_r, dc, w_r, b_r, o_r):
        d2 = dc[...] * dc[...]
        xc = a0[...] + a1[...] + d2 * x_r[...]
        h = jnp.dot(xc, w_r[...], preferred_element_type=jnp.float32) + b_r[...]
        nrm = jnp.sqrt(jnp.sum(h * h, axis=1, keepdims=True))
        h = h / jnp.maximum(nrm, 1e-12)
        o_r[...] = jnp.maximum(h, 0.0)

    fo = W1.shape[1]
    return pl.pallas_call(
        body,
        grid=(N // BR,),
        in_specs=[
            _row_block_specs(FC), _row_block_specs(FC), _row_block_specs(FC),
            pl.BlockSpec((BR, 1), lambda i: (i, 0)),
            pl.BlockSpec((FC, fo), lambda i: (0, 0)),
            pl.BlockSpec((1, fo), lambda i: (0, 0)),
        ],
        out_specs=pl.BlockSpec((BR, fo), lambda i: (i, 0)),
        out_shape=jax.ShapeDtypeStruct((N, fo), jnp.float32),
    )(acc0, acc1, x, dis_col, W1, b1)


def _mm(h, W):
    fi, fo = W.shape

    def body(h_r, w_r, o_r):
        o_r[...] = jnp.dot(h_r[...], w_r[...], preferred_element_type=jnp.float32)

    return pl.pallas_call(
        body,
        grid=(N // BR,),
        in_specs=[
            _row_block_specs(fi),
            pl.BlockSpec((fi, fo), lambda i: (0, 0)),
        ],
        out_specs=pl.BlockSpec((BR, fo), lambda i: (i, 0)),
        out_shape=jax.ShapeDtypeStruct((N, fo), jnp.float32),
    )(h, W)


def _comb(acc0, acc1, t, dis_col, b):
    fo = t.shape[1]

    def body(a0, a1, t_r, dc, b_r, o_r):
        d2 = dc[...] * dc[...]
        h = a0[...] + a1[...] + d2 * t_r[...] + b_r[...]
        nrm = jnp.sqrt(jnp.sum(h * h, axis=1, keepdims=True))
        h = h / jnp.maximum(nrm, 1e-12)
        o_r[...] = jnp.maximum(h, 0.0)

    return pl.pallas_call(
        body,
        grid=(N // BR,),
        in_specs=[
            _row_block_specs(fo), _row_block_specs(fo), _row_block_specs(fo),
            pl.BlockSpec((BR, 1), lambda i: (i, 0)),
            pl.BlockSpec((1, fo), lambda i: (0, 0)),
        ],
        out_specs=pl.BlockSpec((BR, fo), lambda i: (i, 0)),
        out_shape=jax.ShapeDtypeStruct((N, fo), jnp.float32),
    )(acc0, acc1, t, dis_col, b)


def _head(h3, Wmu, bmu, Wlv, blv, eps, beta_arr):
    fi = h3.shape[1]
    fo = Wmu.shape[1]

    def body(h_r, wm, bm, wl, bl, e_r, bet, mu_r, lv_r, pool_r):
        i = pl.program_id(0)
        mu = jnp.dot(h_r[...], wm[...], preferred_element_type=jnp.float32) + bm[...]
        lv = jnp.dot(h_r[...], wl[...], preferred_element_type=jnp.float32) + bl[...]
        mu_r[...] = mu
        lv_r[...] = lv
        std = jnp.exp(0.5 * (bet[0, 0] * lv))
        z = mu + e_r[...] * std
        bmax = jnp.max(z, axis=0, keepdims=True)
        bsum = jnp.sum(z, axis=0, keepdims=True)

        @pl.when(i == 0)
        def _():
            pool_r[0:1, :] = bmax
            pool_r[1:2, :] = bsum

        @pl.when(i > 0)
        def _():
            pool_r[0:1, :] = jnp.maximum(pool_r[0:1, :], bmax)
            pool_r[1:2, :] = pool_r[1:2, :] + bsum

    return pl.pallas_call(
        body,
        grid=(N // BR,),
        in_specs=[
            _row_block_specs(fi),
            pl.BlockSpec((fi, fo), lambda i: (0, 0)),
            pl.BlockSpec((1, fo), lambda i: (0, 0)),
            pl.BlockSpec((fi, fo), lambda i: (0, 0)),
            pl.BlockSpec((1, fo), lambda i: (0, 0)),
            _row_block_specs(fo),
            pl.BlockSpec(memory_space=pltpu.SMEM),
        ],
        out_specs=[
            pl.BlockSpec((BR, fo), lambda i: (i, 0)),
            pl.BlockSpec((BR, fo), lambda i: (i, 0)),
            pl.BlockSpec((2, fo), lambda i: (0, 0)),
        ],
        out_shape=[
            jax.ShapeDtypeStruct((N, fo), jnp.float32),
            jax.ShapeDtypeStruct((N, fo), jnp.float32),
            jax.ShapeDtypeStruct((2, fo), jnp.float32),
        ],
    )(h3, Wmu, bmu, Wlv, blv, eps, beta_arr)


def _dec(pool, Wd1, bd1, Wd2, bd2):
    def body(p_r, w1, b1_r, w2, b2_r, o_r):
        zmax = p_r[0:1, :]
        zmean = p_r[1:2, :] * (1.0 / N)
        rz = jnp.concatenate([zmax, zmean], axis=1)
        h = jnp.dot(rz, w1[...], preferred_element_type=jnp.float32) + b1_r[...]
        h = jnp.maximum(h, 0.0)
        o = jnp.dot(h, w2[...], preferred_element_type=jnp.float32) + b2_r[...]
        o_r[...] = jax.nn.sigmoid(o)

    return pl.pallas_call(
        body, out_shape=jax.ShapeDtypeStruct((1, Wd2.shape[1]), jnp.float32)
    )(pool, Wd1, bd1, Wd2, bd2)


def kernel(x, edge_weight, W1, b1, W2, b2, W3, b3, Wmu, bmu, Wlv, blv,
           Wd1, bd1, Wd2, bd2, edge_index, beta):
    src = edge_index[0].reshape(NW, NB, K)
    dst = edge_index[1].reshape(NW, NB, K)
    eww = edge_weight.reshape(NW, NB, K)

    deg2 = _deg_kernel(dst, eww)
    dis_row = _dis(deg2)
    dis_flat = dis_row.reshape(NP)
    dis_col = dis_flat[:N].reshape(N, 1)
    nrm = _norm_kernel(src, dst, eww, dis_flat)

    src4 = src.reshape(NW, SLABS, SB, K)
    dst4 = dst.reshape(NW, SLABS, SB, K)
    nrm4 = nrm.reshape(NW, SLABS, SB, K)

    acc1 = _spmv_kernel(x, src4, dst4, nrm4)[:, :N, :]
    h1 = _mm1(acc1[0], acc1[1], x, dis_col, W1, b1.reshape(1, -1))

    t2 = _mm(h1, W2)
    acc2 = jnp.concatenate(
        [_spmv_kernel(t2[:, i * FC:(i + 1) * FC], src4, dst4, nrm4)[:, :N, :]
         for i in range(4)], axis=2)
    h2 = _comb(acc2[0], acc2[1], t2, dis_col, b2.reshape(1, -1))

    t3 = _mm(h2, W3)
    acc3 = jnp.concatenate(
        [_spmv_kernel(t3[:, i * FC:(i + 1) * FC], src4, dst4, nrm4)[:, :N, :]
         for i in range(2)], axis=2)
    h3 = _comb(acc3[0], acc3[1], t3, dis_col, b3.reshape(1, -1))

    beta_arr = jnp.asarray(beta, jnp.float32).reshape(1, 1)
    eps = jax.random.normal(jax.random.key(42), (N, 512), jnp.float32) * 0.01
    mu, logvar, pool = _head(h3, Wmu, bmu.reshape(1, -1), Wlv, blv.reshape(1, -1),
                             eps, beta_arr)
    recon = _dec(pool, Wd1, bd1.reshape(1, -1), Wd2, bd2.reshape(1, -1))
    return (recon, mu, logvar)


# speedup vs baseline: 15.7959x; 1.0100x over previous
import functools

import numpy as np
import jax
import jax.numpy as jnp
from jax import lax
from jax.experimental import pallas as pl
from jax.experimental.pallas import tpu as pltpu
from jax.experimental.pallas import tpu_sc as plsc

N = 10000
E = 320000
NP = 10240
NC, NS = 2, 16
NW = NC * NS
K = 80
NB = E // (NW * K)
RT = NP // NS
FC = 128
BR = 1000
SLABS = 5
SB = NB // SLABS


_MESH = plsc.VectorSubcoreMesh(
    core_axis_name="c", subcore_axis_name="s", num_cores=NC, num_subcores=NS)


@functools.partial(
    pl.kernel,
    out_type=jax.ShapeDtypeStruct((NC, NP), jnp.float32),
    mesh=_MESH,
    compiler_params=pltpu.CompilerParams(needs_layout_passes=False),
    scratch_types=[
        pltpu.VMEM((NB, K), jnp.int32),
        pltpu.VMEM((NB, K), jnp.float32),
        pltpu.VMEM((NP // NS,), jnp.float32),
        pltpu.VMEM_SHARED((NP,), jnp.float32),
    ],
)
def _deg_kernel(dst_hbm, ew_hbm, out_hbm, dst_v, ew_v, tbuf, acc):
    c = lax.axis_index("c")
    s = lax.axis_index("s")
    w = s * NC + c
    pltpu.sync_copy(dst_hbm.at[w], dst_v)
    pltpu.sync_copy(ew_hbm.at[w], ew_v)

    def zb(i, _):
        tbuf[pl.ds(i * 16, 16)] = jnp.zeros((16,), jnp.float32)
        return 0

    lax.fori_loop(0, (NP // NS) // 16, zb, 0)
    pltpu.sync_copy(tbuf, acc.at[pl.ds(s * (NP // NS), NP // NS)])
    plsc.subcore_barrier()

    def batch(j, _):
        pltpu.sync_copy(ew_v.at[j], acc.at[dst_v.at[j]], add=True)
        return 0

    lax.fori_loop(0, NB, batch, 0)
    plsc.subcore_barrier()
    pltpu.sync_copy(acc.at[pl.ds(s * (NP // NS), NP // NS)], tbuf)
    pltpu.sync_copy(tbuf, out_hbm.at[c, pl.ds(s * (NP // NS), NP // NS)])


@functools.partial(
    pl.kernel,
    out_type=jax.ShapeDtypeStruct((NW, NB, K), jnp.float32),
    mesh=_MESH,
    compiler_params=pltpu.CompilerParams(needs_layout_passes=False),
    scratch_types=[
        pltpu.VMEM((NB, K), jnp.int32),
        pltpu.VMEM((NB, K), jnp.int32),
        pltpu.VMEM((NB, K), jnp.float32),
        pltpu.VMEM((NB, K), jnp.float32),
        pltpu.VMEM((NP,), jnp.float32),
    ],
)
def _norm_kernel(src_hbm, dst_hbm, ew_hbm, dis_hbm, out_hbm,
                 src_v, dst_v, ew_v, nrm_v, dis_v):
    c = lax.axis_index("c")
    s = lax.axis_index("s")
    w = s * NC + c
    pltpu.sync_copy(src_hbm.at[w], src_v)
    pltpu.sync_copy(dst_hbm.at[w], dst_v)
    pltpu.sync_copy(ew_hbm.at[w], ew_v)
    pltpu.sync_copy(dis_hbm, dis_v)

    def batch(j, _):
        for t in range(K // 16):
            sl = pl.ds(t * 16, 16)
            g1 = plsc.load_gather(dis_v, [src_v[j, sl]])
            g2 = plsc.load_gather(dis_v, [dst_v[j, sl]])
            nrm_v[j, sl] = ew_v[j, sl] * g1 * g2
        return 0

    lax.fori_loop(0, NB, batch, 0)
    pltpu.sync_copy(nrm_v, out_hbm.at[w])


@functools.partial(
    pl.kernel,
    out_type=jax.ShapeDtypeStruct((NC, NP, FC), jnp.float32),
    mesh=_MESH,
    compiler_params=pltpu.CompilerParams(needs_layout_passes=False),
    scratch_types=[
        pltpu.VMEM((SB, K), jnp.int32),
        pltpu.VMEM((SB, K), jnp.int32),
        pltpu.VMEM((SB, K), jnp.float32),
        pltpu.VMEM((K, FC), jnp.float32),
        pltpu.VMEM((K, FC), jnp.float32),
        pltpu.VMEM((K, FC), jnp.float32),
        pltpu.VMEM((RT // 20, FC), jnp.float32),
        pltpu.VMEM_SHARED((NP, FC), jnp.float32),
        pltpu.SemaphoreType.DMA,
        pltpu.SemaphoreType.DMA,
        pltpu.SemaphoreType.DMA,
        pltpu.SemaphoreType.DMA,
        pltpu.SemaphoreType.DMA,
        pltpu.SemaphoreType.DMA,
    ],
)
def _spmv_kernel(x_hbm, src_hbm, dst_hbm, nrm_hbm, out_hbm,
                 src_v, dst_v, nrm_v, r0, r1, r2, wbuf, acc,
                 g0, g1, g2, s0, s1, s2):
    c = lax.axis_index("c")
    s = lax.axis_index("s")
    w = s * NC + c
    rows = (r0, r1, r2)
    gsem = (g0, g1, g2)
    ssem = (s0, s1, s2)

    def zb(i, _):
        for t in range(FC // 16):
            wbuf[i, pl.ds(t * 16, 16)] = jnp.zeros((16,), jnp.float32)
        return 0

    lax.fori_loop(0, RT // 20, zb, 0)
    for q in range(20):
        pltpu.sync_copy(wbuf, acc.at[pl.ds(s * RT + q * (RT // 20), RT // 20)])
    plsc.subcore_barrier()

    def scale(j, buf):
        def scale16(g, _):
            nv16 = nrm_v[j, pl.ds(g * 16, 16)]
            for l in range(16):
                nv = nv16[l]
                k = g * 16 + l
                for t in range(FC // 16):
                    sl = pl.ds(t * 16, 16)
                    buf[k, sl] = buf[k, sl] * nv
            return 0

        lax.fori_loop(0, K // 16, scale16, 0)

    def slab(m, _):
        pltpu.sync_copy(src_hbm.at[w, m], src_v)
        pltpu.sync_copy(dst_hbm.at[w, m], dst_v)
        pltpu.sync_copy(nrm_hbm.at[w, m], nrm_v)
        pltpu.async_copy(x_hbm.at[src_v.at[0]], r0, g0)
        pltpu.async_copy(x_hbm.at[src_v.at[1]], r1, g1)

        def super_batch(q, _):
            for b in range(3):
                j = q * 3 + b
                buf = rows[b]
                nb = (b + 2) % 3
                pltpu.make_async_copy(x_hbm.at[src_v.at[j]], buf, gsem[b]).wait()
                nxt = j + 2

                def prefetch(wait_scatter):
                    if wait_scatter:
                        pltpu.make_async_copy(
                            rows[nb], acc.at[dst_v.at[j]], ssem[nb]).wait()
                    pltpu.async_copy(x_hbm.at[src_v.at[nxt]], rows[nb], gsem[nb])

                if b == 0:
                    @pl.when(q > 0)
                    def _():
                        prefetch(True)

                    @pl.when(q == 0)
                    def _():
                        prefetch(False)
                elif b == 1:
                    prefetch(True)
                else:
                    @pl.when(q < (SB // 3) - 1)
                    def _():
                        prefetch(True)

                scale(j, buf)
                pltpu.async_copy(buf, acc.at[dst_v.at[j]], ssem[b], add=True)
            return 0

        lax.fori_loop(0, SB // 3, super_batch, 0)
        pltpu.make_async_copy(x_hbm.at[src_v.at[SB - 1]], r0, g0).wait()
        scale(SB - 1, r0)
        pltpu.async_copy(r0, acc.at[dst_v.at[SB - 1]], s0, add=True)
        pltpu.make_async_copy(r1, acc.at[dst_v.at[0]], s1).wait()
        pltpu.make_async_copy(r2, acc.at[dst_v.at[0]], s2).wait()
        pltpu.make_async_copy(r0, acc.at[dst_v.at[0]], s0).wait()
        return 0

    lax.fori_loop(0, SLABS, slab, 0)
    plsc.subcore_barrier()
    for q in range(20):
        pltpu.sync_copy(acc.at[pl.ds(s * RT + q * (RT // 20), RT // 20)], wbuf)
        pltpu.sync_copy(wbuf, out_hbm.at[c, pl.ds(s * RT + q * (RT // 20), RT // 20)])


def _dis(deg2):
    def body(deg_ref, out_ref):
        d = deg_ref[0, :] + deg_ref[1, :] + 1.0
        out_ref[0, :] = lax.rsqrt(d)

    return pl.pallas_call(
        body, out_shape=jax.ShapeDtypeStruct((1, NP), jnp.float32)
    )(deg2)


def _row_block_specs(width):
    return pl.BlockSpec((BR, width), lambda i: (i, 0))


def _mm1(acc0, acc1, x, dis_col, W1, b1):
    def body(a0, a1, x_r, dc, w_r, b_r, o_r):
        d2 = dc[...] * dc[...]
        xc = a0[...] + a1[...] + d2 * x_r[...]
        h = jnp.dot(xc, w_r[...], preferred_element_type=jnp.float32) + b_r[...]
        nrm = jnp.sqrt(jnp.sum(h * h, axis=1, keepdims=True))
        h = h / jnp.maximum(nrm, 1e-12)
        o_r[...] = jnp.maximum(h, 0.0)

    fo = W1.shape[1]
    return pl.pallas_call(
        body,
        grid=(N // BR,),
        in_specs=[
            _row_block_specs(FC), _row_block_specs(FC), _row_block_specs(FC),
            pl.BlockSpec((BR, 1), lambda i: (i, 0)),
            pl.BlockSpec((FC, fo), lambda i: (0, 0)),
            pl.BlockSpec((1, fo), lambda i: (0, 0)),
        ],
        out_specs=pl.BlockSpec((BR, fo), lambda i: (i, 0)),
        out_shape=jax.ShapeDtypeStruct((N, fo), jnp.float32),
    )(acc0, acc1, x, dis_col, W1, b1)


def _mm(h, W):
    fi, fo = W.shape

    def body(h_r, w_r, o_r):
        o_r[...] = jnp.dot(h_r[...], w_r[...], preferred_element_type=jnp.float32)

    return pl.pallas_call(
        body,
        grid=(N // BR,),
        in_specs=[
            _row_block_specs(fi),
            pl.BlockSpec((fi, fo), lambda i: (0, 0)),
        ],
        out_specs=pl.BlockSpec((BR, fo), lambda i: (i, 0)),
        out_shape=jax.ShapeDtypeStruct((N, fo), jnp.float32),
    )(h, W)


def _comb(acc0, acc1, t, dis_col, b):
    fo = t.shape[1]

    def body(a0, a1, t_r, dc, b_r, o_r):
        d2 = dc[...] * dc[...]
        h = a0[...] + a1[...] + d2 * t_r[...] + b_r[...]
        nrm = jnp.sqrt(jnp.sum(h * h, axis=1, keepdims=True))
        h = h / jnp.maximum(nrm, 1e-12)
        o_r[...] = jnp.maximum(h, 0.0)

    return pl.pallas_call(
        body,
        grid=(N // BR,),
        in_specs=[
            _row_block_specs(fo), _row_block_specs(fo), _row_block_specs(fo),
            pl.BlockSpec((BR, 1), lambda i: (i, 0)),
            pl.BlockSpec((1, fo), lambda i: (0, 0)),
        ],
        out_specs=pl.BlockSpec((BR, fo), lambda i: (i, 0)),
        out_shape=jax.ShapeDtypeStruct((N, fo), jnp.float32),
    )(acc0, acc1, t, dis_col, b)


def _head(h3, Wmu, bmu, Wlv, blv, eps, beta_arr):
    fi = h3.shape[1]
    fo = Wmu.shape[1]

    def body(h_r, wm, bm, wl, bl, e_r, bet, mu_r, lv_r, pool_r):
        i = pl.program_id(0)
        mu = jnp.dot(h_r[...], wm[...], preferred_element_type=jnp.float32) + bm[...]
        lv = jnp.dot(h_r[...], wl[...], preferred_element_type=jnp.float32) + bl[...]
        mu_r[...] = mu
        lv_r[...] = lv
        std = jnp.exp(0.5 * (bet[0, 0] * lv))
        z = mu + e_r[...] * std
        bmax = jnp.max(z, axis=0, keepdims=True)
        bsum = jnp.sum(z, axis=0, keepdims=True)

        @pl.when(i == 0)
        def _():
            pool_r[0:1, :] = bmax
            pool_r[1:2, :] = bsum

        @pl.when(i > 0)
        def _():
            pool_r[0:1, :] = jnp.maximum(pool_r[0:1, :], bmax)
            pool_r[1:2, :] = pool_r[1:2, :] + bsum

    return pl.pallas_call(
        body,
        grid=(N // BR,),
        in_specs=[
            _row_block_specs(fi),
            pl.BlockSpec((fi, fo), lambda i: (0, 0)),
            pl.BlockSpec((1, fo), lambda i: (0, 0)),
            pl.BlockSpec((fi, fo), lambda i: (0, 0)),
            pl.BlockSpec((1, fo), lambda i: (0, 0)),
            _row_block_specs(fo),
            pl.BlockSpec(memory_space=pltpu.SMEM),
        ],
        out_specs=[
            pl.BlockSpec((BR, fo), lambda i: (i, 0)),
            pl.BlockSpec((BR, fo), lambda i: (i, 0)),
            pl.BlockSpec((2, fo), lambda i: (0, 0)),
        ],
        out_shape=[
            jax.ShapeDtypeStruct((N, fo), jnp.float32),
            jax.ShapeDtypeStruct((N, fo), jnp.float32),
            jax.ShapeDtypeStruct((2, fo), jnp.float32),
        ],
    )(h3, Wmu, bmu, Wlv, blv, eps, beta_arr)


def _dec(pool, Wd1, bd1, Wd2, bd2):
    def body(p_r, w1, b1_r, w2, b2_r, o_r):
        zmax = p_r[0:1, :]
        zmean = p_r[1:2, :] * (1.0 / N)
        rz = jnp.concatenate([zmax, zmean], axis=1)
        h = jnp.dot(rz, w1[...], preferred_element_type=jnp.float32) + b1_r[...]
        h = jnp.maximum(h, 0.0)
        o = jnp.dot(h, w2[...], preferred_element_type=jnp.float32) + b2_r[...]
        o_r[...] = jax.nn.sigmoid(o)

    return pl.pallas_call(
        body, out_shape=jax.ShapeDtypeStruct((1, Wd2.shape[1]), jnp.float32)
    )(pool, Wd1, bd1, Wd2, bd2)


def kernel(x, edge_weight, W1, b1, W2, b2, W3, b3, Wmu, bmu, Wlv, blv,
           Wd1, bd1, Wd2, bd2, edge_index, beta):
    src = edge_index[0].reshape(NW, NB, K)
    dst = edge_index[1].reshape(NW, NB, K)
    eww = edge_weight.reshape(NW, NB, K)

    deg2 = _deg_kernel(dst, eww)
    dis_row = _dis(deg2)
    dis_flat = dis_row.reshape(NP)
    dis_col = dis_flat[:N].reshape(N, 1)
    nrm = _norm_kernel(src, dst, eww, dis_flat)

    src4 = src.reshape(NW, SLABS, SB, K)
    dst4 = dst.reshape(NW, SLABS, SB, K)
    nrm4 = nrm.reshape(NW, SLABS, SB, K)

    acc1 = _spmv_kernel(x, src4, dst4, nrm4)[:, :N, :]
    h1 = _mm1(acc1[0], acc1[1], x, dis_col, W1, b1.reshape(1, -1))

    t2 = _mm(h1, W2)
    acc2 = jnp.concatenate(
        [_spmv_kernel(t2[:, i * FC:(i + 1) * FC], src4, dst4, nrm4)[:, :N, :]
         for i in range(4)], axis=2)
    h2 = _comb(acc2[0], acc2[1], t2, dis_col, b2.reshape(1, -1))

    t3 = _mm(h2, W3)
    acc3 = jnp.concatenate(
        [_spmv_kernel(t3[:, i * FC:(i + 1) * FC], src4, dst4, nrm4)[:, :N, :]
         for i in range(2)], axis=2)
    h3 = _comb(acc3[0], acc3[1], t3, dis_col, b3.reshape(1, -1))

    beta_arr = jnp.asarray(beta, jnp.float32).reshape(1, 1)
    eps = jax.random.normal(jax.random.key(42), (N, 512), jnp.float32) * 0.01
    mu, logvar, pool = _head(h3, Wmu, bmu.reshape(1, -1), Wlv, blv.reshape(1, -1),
                             eps, beta_arr)
    recon = _dec(pool, Wd1, bd1.reshape(1, -1), Wd2, bd2.reshape(1, -1))
    return (recon, mu, logvar)
